# baseline (speedup 1.0000x reference)
import jax
import jax.numpy as jnp
from jax import lax
from jax.experimental import pallas as pl
from jax.experimental.pallas import tpu as pltpu
from jax.experimental.pallas import tpu_sc as plsc

_ROW = 32
_NW = 32
_CH = 256
_G = _CH // 16
_OB = 2048


def _batcher_pairs(n):
    pairs = []

    def merge(lo, m, r):
        step = r * 2
        if step < m:
            merge(lo, m, step)
            merge(lo + r, m, step)
            for i in range(lo + r, lo + m - r, step):
                pairs.append((i, i + r))
        else:
            pairs.append((lo, lo + r))

    def sort(lo, m):
        if m > 1:
            k = m // 2
            sort(lo, k)
            sort(lo + k, k)
            merge(lo, m, 1)

    sort(0, n)
    return pairs


_PAIRS = _batcher_pairs(_ROW)


def _mode16(buf, g, lane):
    rows = lane + g * 16
    vs = [plsc.load_gather(buf, [rows, jnp.full((16,), k, jnp.int32)])
          for k in range(_ROW)]
    for (i, j) in _PAIRS:
        a, b = vs[i], vs[j]
        vs[i] = jnp.minimum(a, b)
        vs[j] = jnp.maximum(a, b)
    anydup = vs[1] == vs[0]
    for k in range(2, _ROW):
        anydup = anydup | (vs[k] == vs[k - 1])

    def with_scan():
        run = jnp.ones((16,), jnp.int32)
        best = run
        bestv = vs[0]
        for k in range(1, _ROW):
            run = run * (vs[k] == vs[k - 1]).astype(jnp.int32) + 1
            bt = run > best
            best = jnp.maximum(run, best)
            bestv = jnp.where(bt, vs[k], bestv)
        return bestv

    return lax.cond(jnp.any(anydup), with_scan, lambda: vs[0])


def _sc_body(x_hbm, o_hbm, buf0, buf1, obuf, sem0, sem1):
    n = o_hbm.shape[0]
    rpw = n // _NW
    nch = rpw // _CH
    wid = lax.axis_index("s") * 2 + lax.axis_index("c")
    base_row = wid * rpw
    lane = lax.iota(jnp.int32, 16)

    def src(c):
        return x_hbm.at[pl.ds(base_row + c * _CH, _CH), :]

    def compute(buf, c):
        ob = (c % (_OB // _CH)) * _CH

        def group(g, _):
            obuf[pl.ds(ob + g * 16, 16)] = _mode16(buf, g, lane)
            return 0

        lax.fori_loop(0, _G, group, 0)

    pltpu.async_copy(src(0), buf0, sem0)
    pltpu.async_copy(src(1), buf1, sem1)

    def pair(cc, _):
        c0 = cc * 2
        pltpu.make_async_copy(src(c0), buf0, sem0).wait()
        compute(buf0, c0)

        @pl.when(cc < nch // 2 - 1)
        def _():
            pltpu.async_copy(src(c0 + 2), buf0, sem0)

        pltpu.make_async_copy(src(c0 + 1), buf1, sem1).wait()
        compute(buf1, c0 + 1)

        @pl.when(cc < nch // 2 - 1)
        def _():
            pltpu.async_copy(src(c0 + 3), buf1, sem1)

        @pl.when((c0 + 2) % (_OB // _CH) == 0)
        def _():
            blk = (c0 + 2) // (_OB // _CH) - 1
            pltpu.sync_copy(obuf, o_hbm.at[pl.ds(base_row + blk * _OB, _OB)])

        return 0

    lax.fori_loop(0, nch // 2, pair, 0)


def kernel(x):
    n = x.shape[0]
    out = pl.kernel(
        _sc_body,
        out_type=jax.ShapeDtypeStruct((n,), jnp.float32),
        mesh=plsc.VectorSubcoreMesh(core_axis_name="c", subcore_axis_name="s"),
        scratch_types=[
            pltpu.VMEM((_CH, _ROW), jnp.float32),
            pltpu.VMEM((_CH, _ROW), jnp.float32),
            pltpu.VMEM((_OB,), jnp.float32),
            pltpu.SemaphoreType.DMA,
            pltpu.SemaphoreType.DMA,
        ],
        compiler_params=pltpu.CompilerParams(needs_layout_passes=False),
    )(x)
    return out


# speedup vs baseline: 1.0000x; 1.0000x over previous
import jax
import jax.numpy as jnp
from jax import lax
from jax.experimental import pallas as pl
from jax.experimental.pallas import tpu as pltpu
from jax.experimental.pallas import tpu_sc as plsc

_ROW = 32
_NW = 32
_CH = 256
_G = _CH // 16
_OB = 2048


def _batcher_pairs(n):
    pairs = []

    def merge(lo, m, r):
        step = r * 2
        if step < m:
            merge(lo, m, step)
            merge(lo + r, m, step)
            for i in range(lo + r, lo + m - r, step):
                pairs.append((i, i + r))
        else:
            pairs.append((lo, lo + r))

    def sort(lo, m):
        if m > 1:
            k = m // 2
            sort(lo, k)
            sort(lo + k, k)
            merge(lo, m, 1)

    sort(0, n)
    return pairs


_PAIRS = _batcher_pairs(_ROW)


def _mode16(buf, g, lane):
    rows = lane + g * 16
    vs = [plsc.load_gather(buf, [rows, jnp.full((16,), k, jnp.int32)])
          for k in range(_ROW)]
    for (i, j) in _PAIRS:
        a, b = vs[i], vs[j]
        vs[i] = jnp.minimum(a, b)
        vs[j] = jnp.maximum(a, b)
    anydup = vs[1] == vs[0]
    for k in range(2, _ROW):
        anydup = anydup | (vs[k] == vs[k - 1])

    def with_scan():
        run = jnp.ones((16,), jnp.int32)
        best = run
        bestv = vs[0]
        for k in range(1, _ROW):
            run = run * (vs[k] == vs[k - 1]).astype(jnp.int32) + 1
            bt = run > best
            best = jnp.maximum(run, best)
            bestv = jnp.where(bt, vs[k], bestv)
        return bestv

    return lax.cond(jnp.any(anydup), with_scan, lambda: vs[0])


def _sc_body(x_hbm, o_hbm, buf0, buf1, obuf, sem0, sem1):
    n = o_hbm.shape[0]
    rpw = n // _NW
    nch = rpw // _CH
    wid = lax.axis_index("s") * 2 + lax.axis_index("c")
    base_row = wid * rpw
    lane = lax.iota(jnp.int32, 16)

    def src(c):
        return x_hbm.at[pl.ds(base_row + c * _CH, _CH), :]

    def compute(buf, c):
        ob = (c % (_OB // _CH)) * _CH

        def group(g, _):
            obuf[pl.ds(ob + g * 16, 16)] = _mode16(buf, g, lane)
            return 0

        lax.fori_loop(0, _G, group, 0)

    pltpu.async_copy(src(0), buf0, sem0)
    pltpu.async_copy(src(1), buf1, sem1)

    def pair(cc, _):
        c0 = cc * 2
        pltpu.make_async_copy(src(c0), buf0, sem0).wait()
        compute(buf0, c0)

        @pl.when(cc < nch // 2 - 1)
        def _():
            pltpu.async_copy(src(c0 + 2), buf0, sem0)

        pltpu.make_async_copy(src(c0 + 1), buf1, sem1).wait()
        compute(buf1, c0 + 1)

        @pl.when(cc < nch // 2 - 1)
        def _():
            pltpu.async_copy(src(c0 + 3), buf1, sem1)

        @pl.when((c0 + 2) % (_OB // _CH) == 0)
        def _():
            blk = (c0 + 2) // (_OB // _CH) - 1
            pltpu.sync_copy(obuf, o_hbm.at[pl.ds(base_row + blk * _OB, _OB)])

        return 0

    lax.fori_loop(0, nch // 2, pair, 0)


def kernel(x):
    n = x.shape[0]
    out = pl.kernel(
        _sc_body,
        out_type=jax.ShapeDtypeStruct((n,), jnp.float32),
        mesh=plsc.VectorSubcoreMesh(core_axis_name="c", subcore_axis_name="s"),
        scratch_types=[
            pltpu.VMEM((_CH, _ROW), jnp.float32),
            pltpu.VMEM((_CH, _ROW), jnp.float32),
            pltpu.VMEM((_OB,), jnp.float32),
            pltpu.SemaphoreType.DMA,
            pltpu.SemaphoreType.DMA,
        ],
        compiler_params=pltpu.CompilerParams(needs_layout_passes=False, use_tc_tiling_on_sc=True),
    )(x)
    return out
